# Initial kernel scaffold; baseline (speedup 1.0000x reference)
import jax
import jax.numpy as jnp
from jax.experimental import pallas as pl


def kernel(base_embed_w, type_embed, reflect, agg_w_self, agg_b_self, agg_w_neigh, agg_b_neigh, vw_q, vw_k, vw_v, vw_fc, vln_g, vln_b, mw_q, mw_k, mw_v, mw_fc, mln_g, mln_b, nodeids, edgetype, neighbors):
    raise NotImplementedError("write your pallas kernel here")


# speedup vs baseline: 14.3229x; 14.3229x over previous
import functools

import jax
import jax.numpy as jnp
import numpy as np
from jax import lax
from jax.experimental import pallas as pl
from jax.experimental.pallas import tpu as pltpu
from jax.experimental.pallas import tpu_sc as plsc

_B = 512
_ED = 128
_NTYPE = 3
_NSCHEMA = 2
_TOT = 18
_NROWS = 3

_NC = 2
_NS = 16
_NW = _NC * _NS
_RPW = _B // _NW

_OFF0 = (0, 16)
_OFF1 = (32, 64)
_OFF2 = (96, 232)


def _build_tidx(nid3, neighbors3):
    nbf = jnp.transpose(neighbors3, (1, 0, 2))
    tcol = jnp.arange(_NTYPE, dtype=jnp.int32)
    z = lambda k: jnp.zeros((k,), jnp.int32)
    idx0, idx1, idx2 = [], [], []
    for s in range(_NSCHEMA):
        toff = tcol[:, None] * 2 + s
        idx0.append((nid3[None, :] * 6 + toff).reshape(-1))
        x1 = nbf[:, :, s * _TOT: s * _TOT + 3]
        idx1.append((x1 * 6 + toff[:, :, None]).reshape(-1))
        x2 = nbf[:, :, s * _TOT + 3: (s + 1) * _TOT]
        idx2.append((x2 * 6 + toff[:, :, None]).reshape(-1))
    return jnp.concatenate([
        idx0[0], z(7), idx0[1], z(7),
        idx1[0], z(5), idx1[1], z(5),
        idx2[0], z(1), idx2[1], z(145),
    ])


def _sc_gather(btab, nid, ttab, tidx):
    mesh = plsc.VectorSubcoreMesh(core_axis_name="c", subcore_axis_name="s")

    @functools.partial(
        pl.kernel,
        mesh=mesh,
        out_type=[jax.ShapeDtypeStruct((_B, 256), jnp.float32),
                  jax.ShapeDtypeStruct((_B, _ED), jnp.float32)],
        scratch_types=[pltpu.VMEM((_RPW,), jnp.int32),
                       pltpu.VMEM((_RPW, 256), jnp.float32),
                       pltpu.VMEM((_RPW,), jnp.int32),
                       pltpu.VMEM((_RPW, _ED), jnp.float32),
                       pltpu.SemaphoreType.DMA,
                       pltpu.SemaphoreType.DMA],
    )
    def k(btab_h, nid_h, ttab_h, tidx_h, bout, tout, nv, brv, tv, trv, sb, st):
        wid = lax.axis_index("s") * _NC + lax.axis_index("c")
        base = wid * _RPW
        pltpu.sync_copy(nid_h.at[pl.ds(base, _RPW)], nv)
        pltpu.sync_copy(tidx_h.at[pl.ds(base, _RPW)], tv)
        cb = pltpu.async_copy(btab_h.at[nv], brv, sb)
        ct = pltpu.async_copy(ttab_h.at[tv], trv, st)
        cb.wait()
        pltpu.sync_copy(brv, bout.at[pl.ds(base, _RPW)])
        ct.wait()
        pltpu.sync_copy(trv, tout.at[pl.ds(base, _RPW)])

    return k(btab, nid, ttab, tidx)


def _mmT(x, w):
    return lax.dot_general(x, w, (((1,), (1,)), ((), ())),
                           preferred_element_type=jnp.float32)


def _mm(x, w):
    return lax.dot_general(x, w, (((1,), (0,)), ((), ())),
                           preferred_element_type=jnp.float32)


def _layer_norm(x, g, b):
    mu = jnp.mean(x, axis=-1, keepdims=True)
    var = jnp.mean((x - mu) * (x - mu), axis=-1, keepdims=True)
    return (x - mu) / jnp.sqrt(var + 1e-6) * g + b


def _seg_mean_mat(groups, size):
    ii = lax.broadcasted_iota(jnp.int32, (groups, groups * size), 0)
    jj = lax.broadcasted_iota(jnp.int32, (groups, groups * size), 1)
    return jnp.where(jj // size == ii, np.float32(1.0 / size), np.float32(0.0))


def _masked_mha(x, wq, wk, wv, wfc, g, b, period):
    n = x.shape[0]
    q = _mmT(_layer_norm(x, g, b), wq) * np.float32(1.0 / np.sqrt(_ED))
    k = _mmT(x, wk)
    v = _mmT(x, wv)
    logits = _mmT(q, k)
    ii = lax.broadcasted_iota(jnp.int32, (n, n), 0)
    jj = lax.broadcasted_iota(jnp.int32, (n, n), 1)
    logits = jnp.where((ii % period) == (jj % period), logits,
                       np.float32(-1e30))
    mx = jnp.max(logits, axis=1, keepdims=True)
    e = jnp.exp(logits - mx)
    a = e / jnp.sum(e, axis=1, keepdims=True)
    return _mmT(_mm(a, v), wfc) + x


def _dense_body(trows_ref, brows_ref, etT_ref, reflect_ref, aws_ref, abs_ref,
                awn_ref, abn_ref, vwq_ref, vwk_ref, vwv_ref, vwfc_ref,
                vlng_ref, vlnb_ref, mwq_ref, mwk_ref, mwv_ref, mwfc_ref,
                mlng_ref, mlnb_ref, out_ref):
    relu = lambda x: jnp.maximum(x, np.float32(0.0))
    trows = trows_ref[...]
    m5 = _seg_mean_mat(27, 5)
    m3 = _seg_mean_mat(9, 3)

    spec = []
    for s in range(_NSCHEMA):
        x0 = trows[_OFF0[s]:_OFF0[s] + 9]
        x1 = trows[_OFF1[s]:_OFF1[s] + 27]
        x2 = trows[_OFF2[s]:_OFF2[s] + 135]
        ws0, ws1 = aws_ref[s, 0], aws_ref[s, 1]
        bs0, bs1 = abs_ref[s, 0], abs_ref[s, 1]
        wn0, wn1 = awn_ref[s, 0], awn_ref[s, 1]
        bn0, bn1 = abn_ref[s, 0], abn_ref[s, 1]
        g1 = relu(jnp.concatenate(
            [_mmT(x1, ws0) + bs0, _mmT(_mm(m5, x2), wn0) + bn0], axis=1))
        g0 = relu(jnp.concatenate(
            [_mmT(x0, ws0) + bs0, _mmT(_mm(m3, x1), wn0) + bn0], axis=1))
        zo = relu(jnp.concatenate(
            [_mmT(g0, ws1) + bs1, _mmT(_mm(m3, g1), wn1) + bn1], axis=1))
        spec.append(_masked_mha(zo, vwq_ref[...], vwk_ref[...], vwv_ref[...],
                                vwfc_ref[...], vlng_ref[...], vlnb_ref[...],
                                period=3))

    z = jnp.concatenate(spec, axis=0)
    z2 = _masked_mha(z, mwq_ref[...], mwk_ref[...], mwv_ref[...],
                     mwfc_ref[...], mlng_ref[...], mlnb_ref[...], period=9)

    pi = lax.broadcasted_iota(jnp.int32, (9, 18), 0)
    pj = lax.broadcasted_iota(jnp.int32, (9, 18), 1)
    mpool = jnp.where((pj % 9) == pi, np.float32(0.5), np.float32(0.0))
    pooled = _mm(mpool, z2)

    tbl = jnp.zeros((9, 256), jnp.float32)
    kk = lax.broadcasted_iota(jnp.int32, (9, 9), 0)
    rr = lax.broadcasted_iota(jnp.int32, (9, 9), 1)
    for t in range(_NTYPE):
        sel = ((kk % 3) == t) & (rr == (t * 3 + kk // 3))
        gt = jnp.where(sel, np.float32(1.0), np.float32(0.0))
        tbl = tbl + _mm(_mm(gt, pooled), reflect_ref[t])

    key = etT_ref[:, 0:1] * 3 + etT_ref[:, 1:2]
    j9 = lax.broadcasted_iota(jnp.int32, (_B, 9), 1)
    oh = jnp.where(key == j9, np.float32(1.0), np.float32(0.0))
    res = brows_ref[...] + _mm(oh, tbl)
    nrm = jnp.maximum(jnp.sqrt(jnp.sum(res * res, axis=1, keepdims=True)),
                      np.float32(1e-12))
    out_ref[...] = res / nrm


def _dense(trows, brows, etT, reflect, aws, ab_s, awn, abn, vwq, vwk, vwv,
           vwfc, vlng, vlnb, mwq, mwk, mwv, mwfc, mlng, mlnb):
    return pl.pallas_call(
        _dense_body,
        out_shape=jax.ShapeDtypeStruct((_B, 256), jnp.float32),
    )(trows, brows, etT, reflect, aws, ab_s, awn, abn, vwq, vwk, vwv,
      vwfc, vlng, vlnb, mwq, mwk, mwv, mwfc, mlng, mlnb)


def kernel(base_embed_w, type_embed, reflect, agg_w_self, agg_b_self,
           agg_w_neigh, agg_b_neigh, vw_q, vw_k, vw_v, vw_fc, vln_g, vln_b,
           mw_q, mw_k, mw_v, mw_fc, mln_g, mln_b, nodeids, edgetype,
           neighbors):
    nid = nodeids.astype(jnp.int32)
    ttab = type_embed.reshape(-1, _ED)
    tidx = _build_tidx(nid[:_NROWS], neighbors[:_NROWS].astype(jnp.int32))
    brows, trows = _sc_gather(base_embed_w, nid, ttab, tidx)
    etT = jnp.transpose(edgetype.astype(jnp.int32))
    return _dense(trows, brows, etT, reflect, agg_w_self, agg_b_self,
                  agg_w_neigh, agg_b_neigh, vw_q, vw_k, vw_v, vw_fc, vln_g,
                  vln_b, mw_q, mw_k, mw_v, mw_fc, mln_g, mln_b)
